# Initial kernel scaffold; baseline (speedup 1.0000x reference)
import jax
import jax.numpy as jnp
from jax.experimental import pallas as pl


def kernel(x_feat, edge_index, bases, W1, b1, W2, b2, W3, b3):
    raise NotImplementedError("write your pallas kernel here")


# speedup vs baseline: 4.2878x; 4.2878x over previous
import functools

import jax
import jax.numpy as jnp
from jax import lax
from jax.experimental import pallas as pl
from jax.experimental.pallas import tpu as pltpu
from jax.experimental.pallas import tpu_sc as plsc

_NC = 2
_NS = 16
_L = 16


def _pre_body(x_ref, w_ref, b_ref, o_ref):
    h = jnp.dot(x_ref[...], w_ref[...], preferred_element_type=jnp.float32)
    h = h + b_ref[...]
    o_ref[...] = 0.5 * h * (1.0 + lax.erf(h * 0.7071067811865476))


def _ffn_body(x_ref, a_ref, w2_ref, b2_ref, w3_ref, b3_ref, o_ref):
    x = x_ref[...] + a_ref[0] + a_ref[1]
    y = jnp.maximum(
        jnp.dot(x, w2_ref[...], preferred_element_type=jnp.float32) + b2_ref[...], 0.0)
    y = jnp.maximum(
        jnp.dot(y, w3_ref[...], preferred_element_type=jnp.float32) + b3_ref[...], 0.0)
    o_ref[...] = x + y


def _pre_ffn(x, W1, b1):
    n, d = x.shape
    r = 1000
    assert n % r == 0
    return pl.pallas_call(
        _pre_body,
        grid=(n // r,),
        in_specs=[
            pl.BlockSpec((r, d), lambda i: (i, 0)),
            pl.BlockSpec((d, d), lambda i: (0, 0)),
            pl.BlockSpec((1, d), lambda i: (0, 0)),
        ],
        out_specs=pl.BlockSpec((r, d), lambda i: (i, 0)),
        out_shape=jax.ShapeDtypeStruct((n, d), jnp.float32),
    )(x, W1, b1.reshape(1, d))


def _final_ffn(x, partials, W2, b2, W3, b3):
    n, d = x.shape
    r = 1000
    assert n % r == 0
    return pl.pallas_call(
        _ffn_body,
        grid=(n // r,),
        in_specs=[
            pl.BlockSpec((r, d), lambda i: (i, 0)),
            pl.BlockSpec((_NC, r, d), lambda i: (0, i, 0)),
            pl.BlockSpec((d, d), lambda i: (0, 0)),
            pl.BlockSpec((1, d), lambda i: (0, 0)),
            pl.BlockSpec((d, d), lambda i: (0, 0)),
            pl.BlockSpec((1, d), lambda i: (0, 0)),
        ],
        out_specs=pl.BlockSpec((r, d), lambda i: (i, 0)),
        out_shape=jax.ShapeDtypeStruct((n, d), jnp.float32),
    )(x, partials, W2, b2.reshape(1, d), W3, b3.reshape(1, d))


@functools.lru_cache(maxsize=None)
def _make_sc_aggr(n, e, d):
    ec = e // _NC
    et = ec // _NS
    c = 80
    assert et % c == 0
    nchunk = et // c
    br = 200
    assert n % br == 0
    nb = n // br
    nbt = -(-nb // _NS)
    nd = d // _L

    mesh = plsc.VectorSubcoreMesh(core_axis_name="c", subcore_axis_name="s")

    @functools.partial(
        pl.kernel,
        mesh=mesh,
        out_type=jax.ShapeDtypeStruct((_NC, n, d), jnp.float32),
        scratch_types=[
            pltpu.VMEM((c,), jnp.int32),
            pltpu.VMEM((c,), jnp.int32),
            pltpu.VMEM((c, d), jnp.float32),
            pltpu.VMEM((c, d), jnp.float32),
            pltpu.VMEM((br, d), jnp.float32),
            pltpu.VMEM_SHARED((n, d), jnp.float32),
            pltpu.SemaphoreType.DMA,
        ],
    )
    def sc_aggr(pre_hbm, src_hbm, dst_hbm, bases_hbm, out_hbm,
                src_v, dst_v, rows_v, bases_v, stage_v, acc_sh, sem):
        cid = lax.axis_index("c")
        sid = lax.axis_index("s")

        def _zrow(i, carry):
            for j in range(nd):
                stage_v[i, pl.ds(j * _L, _L)] = jnp.zeros((_L,), jnp.float32)
            return carry
        lax.fori_loop(0, br, _zrow, 0)
        for t in range(nbt):
            blk = sid + t * _NS
            @pl.when(blk < nb)
            def _():
                r0 = pl.multiple_of(blk * br, 8)
                pltpu.sync_copy(stage_v, acc_sh.at[pl.ds(r0, br)])
        plsc.subcore_barrier()

        ebase = cid * ec + sid * et

        def _chunk(k, carry):
            off = pl.multiple_of(ebase + k * c, 8)
            pltpu.sync_copy(src_hbm.at[pl.ds(off, c)], src_v)
            gat = pltpu.async_copy(pre_hbm.at[src_v], rows_v, sem)
            pltpu.sync_copy(dst_hbm.at[pl.ds(off, c)], dst_v)
            pltpu.sync_copy(bases_hbm.at[pl.ds(off, c)], bases_v)
            gat.wait()

            def _mul(i, icarry):
                for j in range(nd):
                    sl = pl.ds(j * _L, _L)
                    rows_v[i, sl] = rows_v[i, sl] * bases_v[i, sl]
                return icarry
            lax.fori_loop(0, c, _mul, 0)

            pltpu.sync_copy(rows_v, acc_sh.at[dst_v], add=True)
            return carry
        lax.fori_loop(0, nchunk, _chunk, 0)

        plsc.subcore_barrier()
        for t in range(nbt):
            blk = sid + t * _NS
            @pl.when(blk < nb)
            def _():
                r0 = pl.multiple_of(blk * br, 8)
                pltpu.sync_copy(acc_sh.at[pl.ds(r0, br)], stage_v)
                pltpu.sync_copy(stage_v, out_hbm.at[cid].at[pl.ds(r0, br)])

    return sc_aggr


def kernel(x_feat, edge_index, bases, W1, b1, W2, b2, W3, b3):
    n, d = x_feat.shape
    e = edge_index.shape[1]
    pre = _pre_ffn(x_feat, W1, b1)
    partials = _make_sc_aggr(n, e, d)(pre, edge_index[0], edge_index[1], bases)
    return _final_ffn(x_feat, partials, W2, b2, W3, b3)


# speedup vs baseline: 7.1666x; 1.6714x over previous
import functools

import jax
import jax.numpy as jnp
from jax import lax
from jax.experimental import pallas as pl
from jax.experimental.pallas import tpu as pltpu
from jax.experimental.pallas import tpu_sc as plsc

_NC = 2
_NS = 16
_L = 16


def _pre_body(x_ref, w_ref, b_ref, o_ref):
    h = jnp.dot(x_ref[...], w_ref[...], preferred_element_type=jnp.float32)
    h = h + b_ref[...]
    o_ref[...] = 0.5 * h * (1.0 + lax.erf(h * 0.7071067811865476))


def _ffn_body(x_ref, a_ref, w2_ref, b2_ref, w3_ref, b3_ref, o_ref):
    x = x_ref[...] + a_ref[0] + a_ref[1]
    y = jnp.maximum(
        jnp.dot(x, w2_ref[...], preferred_element_type=jnp.float32) + b2_ref[...], 0.0)
    y = jnp.maximum(
        jnp.dot(y, w3_ref[...], preferred_element_type=jnp.float32) + b3_ref[...], 0.0)
    o_ref[...] = x + y


def _pre_ffn(x, W1, b1):
    n, d = x.shape
    r = 1000
    assert n % r == 0
    return pl.pallas_call(
        _pre_body,
        grid=(n // r,),
        in_specs=[
            pl.BlockSpec((r, d), lambda i: (i, 0)),
            pl.BlockSpec((d, d), lambda i: (0, 0)),
            pl.BlockSpec((1, d), lambda i: (0, 0)),
        ],
        out_specs=pl.BlockSpec((r, d), lambda i: (i, 0)),
        out_shape=jax.ShapeDtypeStruct((n, d), jnp.float32),
    )(x, W1, b1.reshape(1, d))


def _final_ffn(x, partials, W2, b2, W3, b3):
    n, d = x.shape
    r = 1000
    assert n % r == 0
    return pl.pallas_call(
        _ffn_body,
        grid=(n // r,),
        in_specs=[
            pl.BlockSpec((r, d), lambda i: (i, 0)),
            pl.BlockSpec((_NC, r, d), lambda i: (0, i, 0)),
            pl.BlockSpec((d, d), lambda i: (0, 0)),
            pl.BlockSpec((1, d), lambda i: (0, 0)),
            pl.BlockSpec((d, d), lambda i: (0, 0)),
            pl.BlockSpec((1, d), lambda i: (0, 0)),
        ],
        out_specs=pl.BlockSpec((r, d), lambda i: (i, 0)),
        out_shape=jax.ShapeDtypeStruct((n, d), jnp.float32),
    )(x, partials, W2, b2.reshape(1, d), W3, b3.reshape(1, d))


@functools.lru_cache(maxsize=None)
def _make_sc_aggr(n, e, d):
    nw = _NC * _NS
    et = e // nw
    c = 40
    assert et % c == 0
    nchunk = et // c
    assert nchunk % 2 == 0
    br = c
    assert n % br == 0
    nb = n // br
    nbt = -(-nb // _NS)
    nd = d // _L

    mesh = plsc.VectorSubcoreMesh(core_axis_name="c", subcore_axis_name="s")

    @functools.partial(
        pl.kernel,
        mesh=mesh,
        out_type=jax.ShapeDtypeStruct((_NC, n, d), jnp.float32),
        scratch_types=[
            pltpu.VMEM((et,), jnp.int32),
            pltpu.VMEM((c,), jnp.int32),
            pltpu.VMEM((c,), jnp.int32),
            pltpu.VMEM((c, d), jnp.float32),
            pltpu.VMEM((c, d), jnp.float32),
            pltpu.VMEM((c, d), jnp.float32),
            pltpu.VMEM((c, d), jnp.float32),
            pltpu.VMEM_SHARED((n, d), jnp.float32),
            pltpu.SemaphoreType.DMA,
            pltpu.SemaphoreType.DMA,
            pltpu.SemaphoreType.DMA,
            pltpu.SemaphoreType.DMA,
            pltpu.SemaphoreType.DMA,
            pltpu.SemaphoreType.DMA,
            pltpu.SemaphoreType.DMA,
        ],
    )
    def sc_aggr(pre_hbm, src_hbm, dst_hbm, bases_hbm, out_hbm,
                srcall_v, dst0, dst1, rows0, rows1, bas0, bas1, acc_sh,
                sem_s, sem_t0, sem_t1, sem_g0, sem_g1, sem_b0, sem_b1):
        cid = lax.axis_index("c")
        sid = lax.axis_index("s")
        wid = cid * _NS + sid
        ebase = wid * et

        cp_s = pltpu.async_copy(
            src_hbm.at[pl.ds(pl.multiple_of(ebase, 8), et)], srcall_v, sem_s)

        def _zrow(i, carry):
            for j in range(nd):
                rows0[i, pl.ds(j * _L, _L)] = jnp.zeros((_L,), jnp.float32)
            return carry
        lax.fori_loop(0, br, _zrow, 0)
        for t in range(nbt):
            blk = sid + t * _NS
            @pl.when(blk < nb)
            def _():
                r0 = pl.multiple_of(blk * br, 8)
                pltpu.sync_copy(rows0, acc_sh.at[pl.ds(r0, br)])
        cp_s.wait()
        plsc.subcore_barrier()

        rowsb = (rows0, rows1)
        basb = (bas0, bas1)
        dstb = (dst0, dst1)
        semg = (sem_g0, sem_g1)
        semb = (sem_b0, sem_b1)
        semt = (sem_t0, sem_t1)

        def _start(k, p):
            off = pl.multiple_of(ebase + k * c, 8)
            pltpu.async_copy(dst_hbm.at[pl.ds(off, c)], dstb[p], semt[p])
            pltpu.async_copy(bases_hbm.at[pl.ds(off, c)], basb[p], semb[p])
            pltpu.async_copy(
                pre_hbm.at[srcall_v.at[pl.ds(k * c, c)]], rowsb[p], semg[p])

        def _finish(k, p):
            off = pl.multiple_of(ebase + k * c, 8)
            pltpu.make_async_copy(bases_hbm.at[pl.ds(off, c)], basb[p], semb[p]).wait()
            pltpu.make_async_copy(
                pre_hbm.at[srcall_v.at[pl.ds(k * c, c)]], rowsb[p], semg[p]).wait()

            def _mul(i, icarry):
                for j in range(nd):
                    sl = pl.ds(j * _L, _L)
                    rowsb[p][i, sl] = rowsb[p][i, sl] * basb[p][i, sl]
                return icarry
            lax.fori_loop(0, c, _mul, 0)

            pltpu.make_async_copy(dst_hbm.at[pl.ds(off, c)], dstb[p], semt[p]).wait()
            pltpu.sync_copy(rowsb[p], acc_sh.at[dstb[p]], add=True)

        _start(0, 0)
        _start(1, 1)

        def _body(g, carry):
            a = g * 2
            _finish(a, 0)
            _start(a + 2, 0)
            _finish(a + 1, 1)
            _start(a + 3, 1)
            return carry
        lax.fori_loop(0, (nchunk - 2) // 2, _body, 0)
        _finish(nchunk - 2, 0)
        _finish(nchunk - 1, 1)

        plsc.subcore_barrier()
        for t in range(nbt):
            blk = sid + t * _NS
            @pl.when(blk < nb)
            def _():
                r0 = pl.multiple_of(blk * br, 8)
                pltpu.sync_copy(acc_sh.at[pl.ds(r0, br)], rows0)
                pltpu.sync_copy(rows0, out_hbm.at[cid].at[pl.ds(r0, br)])

    return sc_aggr


def kernel(x_feat, edge_index, bases, W1, b1, W2, b2, W3, b3):
    n, d = x_feat.shape
    e = edge_index.shape[1]
    pre = _pre_ffn(x_feat, W1, b1)
    partials = _make_sc_aggr(n, e, d)(
        pre, edge_index[0], edge_index[1], bases)
    return _final_ffn(x_feat, partials, W2, b2, W3, b3)


# speedup vs baseline: 7.5276x; 1.0504x over previous
import functools

import jax
import jax.numpy as jnp
from jax import lax
from jax.experimental import pallas as pl
from jax.experimental.pallas import tpu as pltpu
from jax.experimental.pallas import tpu_sc as plsc

_NC = 2
_NS = 16
_L = 16


def _pre_body(x_ref, w_ref, b_ref, o_ref):
    h = jnp.dot(x_ref[...], w_ref[...], preferred_element_type=jnp.float32)
    h = h + b_ref[...]
    o_ref[...] = 0.5 * h * (1.0 + lax.erf(h * 0.7071067811865476))


def _ffn_body(x_ref, a_ref, w2_ref, b2_ref, w3_ref, b3_ref, o_ref):
    x = x_ref[...] + a_ref[0] + a_ref[1]
    y = jnp.maximum(
        jnp.dot(x, w2_ref[...], preferred_element_type=jnp.float32) + b2_ref[...], 0.0)
    y = jnp.maximum(
        jnp.dot(y, w3_ref[...], preferred_element_type=jnp.float32) + b3_ref[...], 0.0)
    o_ref[...] = x + y


def _pre_ffn(x, W1, b1):
    n, d = x.shape
    r = 1000
    assert n % r == 0
    return pl.pallas_call(
        _pre_body,
        grid=(n // r,),
        in_specs=[
            pl.BlockSpec((r, d), lambda i: (i, 0)),
            pl.BlockSpec((d, d), lambda i: (0, 0)),
            pl.BlockSpec((1, d), lambda i: (0, 0)),
        ],
        out_specs=pl.BlockSpec((r, d), lambda i: (i, 0)),
        out_shape=jax.ShapeDtypeStruct((n, d), jnp.float32),
    )(x, W1, b1.reshape(1, d))


def _final_ffn(x, partials, W2, b2, W3, b3):
    n, d = x.shape
    r = 1000
    assert n % r == 0
    return pl.pallas_call(
        _ffn_body,
        grid=(n // r,),
        in_specs=[
            pl.BlockSpec((r, d), lambda i: (i, 0)),
            pl.BlockSpec((_NC, r, d), lambda i: (0, i, 0)),
            pl.BlockSpec((d, d), lambda i: (0, 0)),
            pl.BlockSpec((1, d), lambda i: (0, 0)),
            pl.BlockSpec((d, d), lambda i: (0, 0)),
            pl.BlockSpec((1, d), lambda i: (0, 0)),
        ],
        out_specs=pl.BlockSpec((r, d), lambda i: (i, 0)),
        out_shape=jax.ShapeDtypeStruct((n, d), jnp.float32),
    )(x, partials, W2, b2.reshape(1, d), W3, b3.reshape(1, d))


@functools.lru_cache(maxsize=None)
def _make_sc_aggr(n, e, d):
    nw = _NC * _NS
    et = e // nw
    c = 40
    assert et % c == 0
    nchunk = et // c
    assert nchunk % 4 == 2
    br = c
    assert n % br == 0
    nb = n // br
    nbt = -(-nb // _NS)
    nd = d // _L

    mesh = plsc.VectorSubcoreMesh(core_axis_name="c", subcore_axis_name="s")

    @functools.partial(
        pl.kernel,
        mesh=mesh,
        out_type=jax.ShapeDtypeStruct((_NC, n, d), jnp.float32),
        scratch_types=[
            pltpu.VMEM((et,), jnp.int32),
            pltpu.VMEM((c,), jnp.int32),
            pltpu.VMEM((c,), jnp.int32),
            pltpu.VMEM((c,), jnp.int32),
            pltpu.VMEM((c,), jnp.int32),
            pltpu.VMEM((c, d), jnp.float32),
            pltpu.VMEM((c, d), jnp.float32),
            pltpu.VMEM((c, d), jnp.float32),
            pltpu.VMEM((c, d), jnp.float32),
            pltpu.VMEM((c, d), jnp.float32),
            pltpu.VMEM((c, d), jnp.float32),
            pltpu.VMEM_SHARED((n, d), jnp.float32),
            pltpu.SemaphoreType.DMA,
            pltpu.SemaphoreType.DMA,
            pltpu.SemaphoreType.DMA,
            pltpu.SemaphoreType.DMA,
            pltpu.SemaphoreType.DMA,
            pltpu.SemaphoreType.DMA,
            pltpu.SemaphoreType.DMA,
            pltpu.SemaphoreType.DMA,
            pltpu.SemaphoreType.DMA,
            pltpu.SemaphoreType.DMA,
            pltpu.SemaphoreType.DMA,
            pltpu.SemaphoreType.DMA,
            pltpu.SemaphoreType.DMA,
            pltpu.SemaphoreType.DMA,
            pltpu.SemaphoreType.DMA,
        ],
    )
    def sc_aggr(pre_hbm, src_hbm, dst_hbm, bases_hbm, out_hbm,
                srcall_v, dst0, dst1, dst2, dst3,
                rows0, rows1, rows2, rows3, bas0, bas1, acc_sh,
                sem_s, sem_t0, sem_t1, sem_t2, sem_t3,
                sem_g0, sem_g1, sem_g2, sem_g3, sem_b0, sem_b1,
                sem_c0, sem_c1, sem_c2, sem_c3):
        cid = lax.axis_index("c")
        sid = lax.axis_index("s")
        wid = cid * _NS + sid
        ebase = wid * et

        cp_s = pltpu.async_copy(
            src_hbm.at[pl.ds(pl.multiple_of(ebase, 8), et)], srcall_v, sem_s)

        def _zrow(i, carry):
            for j in range(nd):
                rows0[i, pl.ds(j * _L, _L)] = jnp.zeros((_L,), jnp.float32)
            return carry
        lax.fori_loop(0, br, _zrow, 0)
        for t in range(nbt):
            blk = sid + t * _NS
            @pl.when(blk < nb)
            def _():
                r0 = pl.multiple_of(blk * br, 8)
                pltpu.sync_copy(rows0, acc_sh.at[pl.ds(r0, br)])
        cp_s.wait()
        plsc.subcore_barrier()

        rowsb = (rows0, rows1, rows2, rows3)
        basb = (bas0, bas1)
        dstb = (dst0, dst1, dst2, dst3)
        semg = (sem_g0, sem_g1, sem_g2, sem_g3)
        semb = (sem_b0, sem_b1)
        semt = (sem_t0, sem_t1, sem_t2, sem_t3)
        semc = (sem_c0, sem_c1, sem_c2, sem_c3)

        def _drain(q):
            pltpu.make_async_copy(rowsb[q], acc_sh.at[dstb[q]], semc[q]).wait()

        def _start(k, q, qb, first=False):
            if not first:
                @pl.when(k >= 4)
                def _():
                    _drain(q)
            off = pl.multiple_of(ebase + k * c, 8)
            pltpu.async_copy(dst_hbm.at[pl.ds(off, c)], dstb[q], semt[q])
            pltpu.async_copy(bases_hbm.at[pl.ds(off, c)], basb[qb], semb[qb])
            pltpu.async_copy(
                pre_hbm.at[srcall_v.at[pl.ds(k * c, c)]], rowsb[q], semg[q])

        def _finish(k, q, qb):
            off = pl.multiple_of(ebase + k * c, 8)
            pltpu.make_async_copy(bases_hbm.at[pl.ds(off, c)], basb[qb], semb[qb]).wait()
            pltpu.make_async_copy(
                pre_hbm.at[srcall_v.at[pl.ds(k * c, c)]], rowsb[q], semg[q]).wait()

            @plsc.parallel_loop(0, c, step=1, unroll=8)
            def _mul(i):
                for j in range(nd):
                    sl = pl.ds(j * _L, _L)
                    rowsb[q][i, sl] = rowsb[q][i, sl] * basb[qb][i, sl]

            pltpu.make_async_copy(dst_hbm.at[pl.ds(off, c)], dstb[q], semt[q]).wait()
            pltpu.async_copy(rowsb[q], acc_sh.at[dstb[q]], semc[q], add=True)

        _start(0, 0, 0, first=True)
        _start(1, 1, 1, first=True)

        def _body(g, carry):
            a = g * 4
            _finish(a, 0, 0)
            _start(a + 2, 2, 0)
            _finish(a + 1, 1, 1)
            _start(a + 3, 3, 1)
            _finish(a + 2, 2, 0)
            _start(a + 4, 0, 0)
            _finish(a + 3, 3, 1)
            _start(a + 5, 1, 1)
            return carry
        lax.fori_loop(0, (nchunk - 2) // 4, _body, 0)
        _finish(nchunk - 2, 0, 0)
        _finish(nchunk - 1, 1, 1)
        _drain(2)
        _drain(3)
        _drain(0)
        _drain(1)

        plsc.subcore_barrier()
        for t in range(nbt):
            blk = sid + t * _NS
            @pl.when(blk < nb)
            def _():
                r0 = pl.multiple_of(blk * br, 8)
                pltpu.sync_copy(acc_sh.at[pl.ds(r0, br)], rows0)
                pltpu.sync_copy(rows0, out_hbm.at[cid].at[pl.ds(r0, br)])

    return sc_aggr


def kernel(x_feat, edge_index, bases, W1, b1, W2, b2, W3, b3):
    n, d = x_feat.shape
    e = edge_index.shape[1]
    pre = _pre_ffn(x_feat, W1, b1)
    partials = _make_sc_aggr(n, e, d)(
        pre, edge_index[0], edge_index[1], bases)
    return _final_ffn(x_feat, partials, W2, b2, W3, b3)
